# baseline (speedup 1.0000x reference)
import functools

import jax
import jax.numpy as jnp
import numpy as np
from jax import lax
from jax.experimental import pallas as pl
from jax.experimental.pallas import tpu as pltpu
from jax.experimental.pallas import tpu_sc as plsc

PS = 7
B, HD, T, C, H, W = 1, 4, 2, 32, 32, 32
K = 10
BH = B * HD
PADW = W + PS - 1
ROWW = PADW * C
VWORDS = T * PADW * ROWW
STRIP_ROWS = 14
STRIP = STRIP_ROWS * ROWW
NTEC = 32
QPT = 256


def _sc_body(vpad_hbm, bq_hbm, dq_hbm, out_hbm, vpad_v, bq_v, dq_v, opad_v):
    cid = lax.axis_index("c")
    sid = lax.axis_index("s")
    wid = sid * 2 + cid
    bh = wid // 8
    pltpu.sync_copy(vpad_hbm.at[bh], vpad_v)
    pltpu.sync_copy(bq_hbm.at[wid], bq_v)
    pltpu.sync_copy(dq_hbm.at[wid], dq_v)

    iota = jnp.arange(16, dtype=jnp.int32)
    iota2 = iota * 2
    zeros = jnp.zeros((16,), jnp.float32)

    def zbody(i, _):
        plsc.store_scatter(opad_v, [i * 16 + iota], zeros)
        return 0

    lax.fori_loop(0, STRIP // 16, zbody, 0)

    @plsc.parallel_loop(0, QPT)
    def qbody(q):
        nh = q // 32
        nw = q - nh * 32
        obase = nh * ROWW + nw * C
        bvec = bq_v[pl.ds(q * K, 16)]
        wvec = dq_v[pl.ds(q * K, 16)]
        bs = [bvec[k] for k in range(K)]
        ws = [
            plsc.bitcast(jnp.full((16,), wvec[k], jnp.int32), jnp.bfloat16)
            for k in range(K)
        ]
        for i in range(PS):
            roff2 = i * (ROWW // 2)
            for m in range(PS):
                off2 = roff2 + m * 16
                vk = plsc.bitcast(vpad_v[pl.ds(bs[0] + off2, 16)], jnp.bfloat16)
                a0 = vk * ws[0]
                vk = plsc.bitcast(vpad_v[pl.ds(bs[1] + off2, 16)], jnp.bfloat16)
                a1 = vk * ws[1]
                for k in range(2, K, 2):
                    vk = plsc.bitcast(
                        vpad_v[pl.ds(bs[k] + off2, 16)], jnp.bfloat16)
                    a0 = a0 + vk * ws[k]
                    vk = plsc.bitcast(
                        vpad_v[pl.ds(bs[k + 1] + off2, 16)], jnp.bfloat16)
                    a1 = a1 + vk * ws[k + 1]
                lo, hi = plsc.unpack(a0 + a1, format=plsc.PackFormat.INTERLEAVED)
                w0 = obase + i * ROWW + m * 32
                plsc.addupdate_scatter(opad_v, [w0 + iota2], lo)
                plsc.addupdate_scatter(opad_v, [w0 + 1 + iota2], hi)
    pltpu.sync_copy(opad_v, out_hbm.at[wid])


_sc_call = functools.partial(
    pl.kernel,
    out_type=jax.ShapeDtypeStruct((NTEC, STRIP), jnp.float32),
    mesh=plsc.VectorSubcoreMesh(core_axis_name="c", subcore_axis_name="s"),
    compiler_params=pltpu.CompilerParams(needs_layout_passes=False),
    scratch_types=[
        pltpu.VMEM((VWORDS // 2,), jnp.int32),
        pltpu.VMEM((QPT * K,), jnp.int32),
        pltpu.VMEM((QPT * K,), jnp.int32),
        pltpu.VMEM((STRIP,), jnp.float32),
    ],
)(_sc_body)


def _tc_body(parts_ref, invc_ref, out_ref):
    p = parts_ref[...]

    def pad_rows(x, lo, hi):
        z = jnp.zeros((8, 1, ROWW), jnp.float32)
        pieces = [z] * lo + [x] + [z] * hi
        return jnp.concatenate(pieces, axis=1) if len(pieces) > 1 else x

    opad = sum(
        pad_rows(p[:, blk], blk * 8, PADW - STRIP_ROWS - blk * 8)
        for blk in range(4)
    )
    rev_rows = jnp.concatenate(
        [opad[:, PADW - 1 - m: PADW - m, :] for m in range(PS - 1)], axis=1
    )
    top = opad[:, :H, :] + pad_rows(rev_rows, 25, 1)
    zc = jnp.zeros((8, H, C), jnp.float32)
    rev_cols = [zc] * 25 + [
        top[:, :, (PADW - 1 - m) * C: (PADW - m) * C] for m in range(PS - 1)
    ] + [zc]
    res = top[:, :, : W * C] + jnp.concatenate(rev_cols, axis=2)
    out_ref[...] = res * invc_ref[...][None, :, :]


def _fold_counts():
    dy = np.arange(PS)
    hh = np.abs(np.arange(H)[:, None] + dy[None, :])
    hh = np.where(hh > H - 1, 2 * (H - 1) - hh, hh)
    ch = np.zeros(H, np.float64)
    np.add.at(ch, hh, 1.0)
    invc = 1.0 / (ch[:, None] * ch[None, :])
    return np.repeat(invc, C, axis=1).astype(np.float32)


_INVC = _fold_counts()


def kernel(vid_in, dists, inds):
    vid = vid_in.reshape(BH, T, C, H, W).transpose(0, 1, 3, 4, 2)
    vpad = jnp.pad(vid, ((0, 0), (0, 0), (0, PS - 1), (0, PS - 1), (0, 0)),
                   mode="reflect")
    vbits = lax.bitcast_convert_type(
        vpad.astype(jnp.bfloat16), jnp.uint16).astype(jnp.uint32)
    vflat = lax.bitcast_convert_type(
        vbits[..., 0::2] | (vbits[..., 1::2] << 16), jnp.int32
    ).reshape(BH, VWORDS // 2)

    ix = inds.reshape(BH, T, H, W, K, 3).astype(jnp.int32)
    tn = ix[..., 0] % T
    bq = ((tn * PADW + ix[..., 1]) * PADW + ix[..., 2]) * (C // 2)
    bq_t = bq.reshape(BH, T, 4, 8, W, K).reshape(NTEC, QPT * K)
    dbits = lax.bitcast_convert_type(
        dists.astype(jnp.bfloat16), jnp.uint16).astype(jnp.uint32)
    dq_t = lax.bitcast_convert_type(
        dbits | (dbits << 16), jnp.int32
    ).reshape(BH, T, 4, 8, W, K).reshape(NTEC, QPT * K)

    parts = _sc_call(vflat, bq_t, dq_t)
    parts = parts.reshape(8, 4, STRIP_ROWS, ROWW)

    res = pl.pallas_call(
        _tc_body,
        out_shape=jax.ShapeDtypeStruct((8, H, W * C), jnp.float32),
    )(parts, jnp.asarray(_INVC))

    out = res.reshape(BH, T, H, W, C).transpose(0, 1, 4, 2, 3)
    return out.reshape(B, HD, T, C, H, W)


# speedup vs baseline: 1.3009x; 1.3009x over previous
import functools

import jax
import jax.numpy as jnp
import numpy as np
from jax import lax
from jax.experimental import pallas as pl
from jax.experimental.pallas import tpu as pltpu
from jax.experimental.pallas import tpu_sc as plsc

PS = 7
B, HD, T, C, H, W = 1, 4, 2, 32, 32, 32
K = 10
BH = B * HD
PADW = W + PS - 1
ROWW = PADW * C
VWORDS = T * PADW * ROWW
STRIP_ROWS = 14
STRIP = STRIP_ROWS * ROWW
NTEC = 32
QPT = 256


def _sc_body(vpad_hbm, bq_hbm, dq_hbm, out_hbm, vpad_v, bq_v, dq_v, opad_v):
    cid = lax.axis_index("c")
    sid = lax.axis_index("s")
    wid = sid * 2 + cid
    bh = wid // 8
    pltpu.sync_copy(vpad_hbm.at[bh], vpad_v)
    pltpu.sync_copy(bq_hbm.at[wid], bq_v)
    pltpu.sync_copy(dq_hbm.at[wid], dq_v)

    iota = jnp.arange(16, dtype=jnp.int32)
    iota2 = iota * 2
    zeros = jnp.zeros((16,), jnp.float32)

    def zbody(i, _):
        plsc.store_scatter(opad_v, [i * 16 + iota], zeros)
        return 0

    lax.fori_loop(0, STRIP // 16, zbody, 0)

    @plsc.parallel_loop(0, QPT * PS)
    def qbody(t):
        q = t // PS
        i = t - q * PS
        nh = q // 32
        nw = q - nh * 32
        obase = nh * ROWW + nw * C
        bvec = bq_v[pl.ds(q * K, 16)]
        wvec = dq_v[pl.ds(q * K, 16)]
        bs = [bvec[k] for k in range(K)]
        ws = [
            plsc.bitcast(jnp.full((16,), wvec[k], jnp.int32), jnp.bfloat16)
            for k in range(K)
        ]
        roff2 = i * (ROWW // 2)
        for m in range(PS):
            off2 = roff2 + m * 16
            vk = plsc.bitcast(vpad_v[pl.ds(bs[0] + off2, 16)], jnp.bfloat16)
            a0 = vk * ws[0]
            vk = plsc.bitcast(vpad_v[pl.ds(bs[1] + off2, 16)], jnp.bfloat16)
            a1 = vk * ws[1]
            for k in range(2, K, 2):
                vk = plsc.bitcast(
                    vpad_v[pl.ds(bs[k] + off2, 16)], jnp.bfloat16)
                a0 = a0 + vk * ws[k]
                vk = plsc.bitcast(
                    vpad_v[pl.ds(bs[k + 1] + off2, 16)], jnp.bfloat16)
                a1 = a1 + vk * ws[k + 1]
            lo, hi = plsc.unpack(a0 + a1, format=plsc.PackFormat.INTERLEAVED)
            w0 = obase + i * ROWW + m * 32
            plsc.addupdate_scatter(opad_v, [w0 + iota2], lo)
            plsc.addupdate_scatter(opad_v, [w0 + 1 + iota2], hi)
    pltpu.sync_copy(opad_v, out_hbm.at[wid])


_sc_call = functools.partial(
    pl.kernel,
    out_type=jax.ShapeDtypeStruct((NTEC, STRIP), jnp.float32),
    mesh=plsc.VectorSubcoreMesh(core_axis_name="c", subcore_axis_name="s"),
    compiler_params=pltpu.CompilerParams(needs_layout_passes=False),
    scratch_types=[
        pltpu.VMEM((VWORDS // 2,), jnp.int32),
        pltpu.VMEM((QPT * K,), jnp.int32),
        pltpu.VMEM((QPT * K,), jnp.int32),
        pltpu.VMEM((STRIP,), jnp.float32),
    ],
)(_sc_body)


def _tc_body(parts_ref, invc_ref, out_ref):
    p = parts_ref[...]

    def pad_rows(x, lo, hi):
        z = jnp.zeros((8, 1, ROWW), jnp.float32)
        pieces = [z] * lo + [x] + [z] * hi
        return jnp.concatenate(pieces, axis=1) if len(pieces) > 1 else x

    opad = sum(
        pad_rows(p[:, blk], blk * 8, PADW - STRIP_ROWS - blk * 8)
        for blk in range(4)
    )
    rev_rows = jnp.concatenate(
        [opad[:, PADW - 1 - m: PADW - m, :] for m in range(PS - 1)], axis=1
    )
    top = opad[:, :H, :] + pad_rows(rev_rows, 25, 1)
    zc = jnp.zeros((8, H, C), jnp.float32)
    rev_cols = [zc] * 25 + [
        top[:, :, (PADW - 1 - m) * C: (PADW - m) * C] for m in range(PS - 1)
    ] + [zc]
    res = top[:, :, : W * C] + jnp.concatenate(rev_cols, axis=2)
    out_ref[...] = res * invc_ref[...][None, :, :]


def _fold_counts():
    dy = np.arange(PS)
    hh = np.abs(np.arange(H)[:, None] + dy[None, :])
    hh = np.where(hh > H - 1, 2 * (H - 1) - hh, hh)
    ch = np.zeros(H, np.float64)
    np.add.at(ch, hh, 1.0)
    invc = 1.0 / (ch[:, None] * ch[None, :])
    return np.repeat(invc, C, axis=1).astype(np.float32)


_INVC = _fold_counts()


def kernel(vid_in, dists, inds):
    vid = vid_in.reshape(BH, T, C, H, W).transpose(0, 1, 3, 4, 2)
    vpad = jnp.pad(vid, ((0, 0), (0, 0), (0, PS - 1), (0, PS - 1), (0, 0)),
                   mode="reflect")
    vbits = lax.bitcast_convert_type(
        vpad.astype(jnp.bfloat16), jnp.uint16).astype(jnp.uint32)
    vflat = lax.bitcast_convert_type(
        vbits[..., 0::2] | (vbits[..., 1::2] << 16), jnp.int32
    ).reshape(BH, VWORDS // 2)

    ix = inds.reshape(BH, T, H, W, K, 3).astype(jnp.int32)
    tn = ix[..., 0] % T
    bq = ((tn * PADW + ix[..., 1]) * PADW + ix[..., 2]) * (C // 2)
    bq_t = bq.reshape(BH, T, 4, 8, W, K).reshape(NTEC, QPT * K)
    dbits = lax.bitcast_convert_type(
        dists.astype(jnp.bfloat16), jnp.uint16).astype(jnp.uint32)
    dq_t = lax.bitcast_convert_type(
        dbits | (dbits << 16), jnp.int32
    ).reshape(BH, T, 4, 8, W, K).reshape(NTEC, QPT * K)

    parts = _sc_call(vflat, bq_t, dq_t)
    parts = parts.reshape(8, 4, STRIP_ROWS, ROWW)

    res = pl.pallas_call(
        _tc_body,
        out_shape=jax.ShapeDtypeStruct((8, H, W * C), jnp.float32),
    )(parts, jnp.asarray(_INVC))

    out = res.reshape(BH, T, H, W, C).transpose(0, 1, 4, 2, 3)
    return out.reshape(B, HD, T, C, H, W)


# speedup vs baseline: 1.3132x; 1.0095x over previous
import functools

import jax
import jax.numpy as jnp
import numpy as np
from jax import lax
from jax.experimental import pallas as pl
from jax.experimental.pallas import tpu as pltpu
from jax.experimental.pallas import tpu_sc as plsc

PS = 7
B, HD, T, C, H, W = 1, 4, 2, 32, 32, 32
K = 10
BH = B * HD
PADW = W + PS - 1
ROWW = PADW * C
VWORDS = T * PADW * ROWW
STRIP_ROWS = 14
STRIP = STRIP_ROWS * ROWW
NTEC = 32
QPT = 256


def _sc_body(vpad_hbm, bq_hbm, dq_hbm, out_hbm, vpad_v, bq_v, dq_v, opad_v,
             sem):
    cid = lax.axis_index("c")
    sid = lax.axis_index("s")
    wid = sid * 2 + cid
    bh = wid // 8
    cp1 = pltpu.make_async_copy(vpad_hbm.at[bh], vpad_v, sem)
    cp2 = pltpu.make_async_copy(bq_hbm.at[wid], bq_v, sem)
    cp3 = pltpu.make_async_copy(dq_hbm.at[wid], dq_v, sem)
    cp1.start()
    cp2.start()
    cp3.start()

    iota = jnp.arange(16, dtype=jnp.int32)
    iota2 = iota * 2
    zeros = jnp.zeros((16,), jnp.float32)

    def zbody(i, _):
        plsc.store_scatter(opad_v, [i * 16 + iota], zeros)
        return 0

    lax.fori_loop(0, STRIP // 16, zbody, 0)
    cp1.wait()
    cp2.wait()
    cp3.wait()

    def qbody(q, _):
        nh = q // 32
        nw = q - nh * 32
        obase = nh * ROWW + nw * C
        bvec = bq_v[pl.ds(q * K, 16)]
        wvec = dq_v[pl.ds(q * K, 16)]
        bs = [bvec[k] for k in range(K)]
        ws = [
            plsc.bitcast(jnp.full((16,), wvec[k], jnp.int32), jnp.bfloat16)
            for k in range(K)
        ]

        @plsc.parallel_loop(0, PS, unroll=2)
        def ibody(i):
            roff2 = i * (ROWW // 2)
            for m in range(PS):
                off2 = roff2 + m * 16
                vk = plsc.bitcast(vpad_v[pl.ds(bs[0] + off2, 16)], jnp.bfloat16)
                a0 = vk * ws[0]
                vk = plsc.bitcast(vpad_v[pl.ds(bs[1] + off2, 16)], jnp.bfloat16)
                a1 = vk * ws[1]
                for k in range(2, K, 2):
                    vk = plsc.bitcast(
                        vpad_v[pl.ds(bs[k] + off2, 16)], jnp.bfloat16)
                    a0 = a0 + vk * ws[k]
                    vk = plsc.bitcast(
                        vpad_v[pl.ds(bs[k + 1] + off2, 16)], jnp.bfloat16)
                    a1 = a1 + vk * ws[k + 1]
                lo, hi = plsc.unpack(a0 + a1, format=plsc.PackFormat.INTERLEAVED)
                w0 = obase + i * ROWW + m * 32
                plsc.addupdate_scatter(opad_v, [w0 + iota2], lo)
                plsc.addupdate_scatter(opad_v, [w0 + 1 + iota2], hi)

        return 0

    lax.fori_loop(0, QPT, qbody, 0)
    pltpu.sync_copy(opad_v, out_hbm.at[wid])


_sc_call = functools.partial(
    pl.kernel,
    out_type=jax.ShapeDtypeStruct((NTEC, STRIP), jnp.float32),
    mesh=plsc.VectorSubcoreMesh(core_axis_name="c", subcore_axis_name="s"),
    compiler_params=pltpu.CompilerParams(needs_layout_passes=False),
    scratch_types=[
        pltpu.VMEM((VWORDS // 2,), jnp.int32),
        pltpu.VMEM((QPT * K,), jnp.int32),
        pltpu.VMEM((QPT * K,), jnp.int32),
        pltpu.VMEM((STRIP,), jnp.float32),
        pltpu.SemaphoreType.DMA,
    ],
)(_sc_body)


def _tc_body(parts_ref, invc_ref, out_ref):
    p = parts_ref[...]

    def pad_rows(x, lo, hi):
        z = jnp.zeros((8, 1, ROWW), jnp.float32)
        pieces = [z] * lo + [x] + [z] * hi
        return jnp.concatenate(pieces, axis=1) if len(pieces) > 1 else x

    opad = sum(
        pad_rows(p[:, blk], blk * 8, PADW - STRIP_ROWS - blk * 8)
        for blk in range(4)
    )
    rev_rows = jnp.concatenate(
        [opad[:, PADW - 1 - m: PADW - m, :] for m in range(PS - 1)], axis=1
    )
    top = opad[:, :H, :] + pad_rows(rev_rows, 25, 1)
    zc = jnp.zeros((8, H, C), jnp.float32)
    rev_cols = [zc] * 25 + [
        top[:, :, (PADW - 1 - m) * C: (PADW - m) * C] for m in range(PS - 1)
    ] + [zc]
    res = top[:, :, : W * C] + jnp.concatenate(rev_cols, axis=2)
    out_ref[...] = res * invc_ref[...][None, :, :]


def _fold_counts():
    dy = np.arange(PS)
    hh = np.abs(np.arange(H)[:, None] + dy[None, :])
    hh = np.where(hh > H - 1, 2 * (H - 1) - hh, hh)
    ch = np.zeros(H, np.float64)
    np.add.at(ch, hh, 1.0)
    invc = 1.0 / (ch[:, None] * ch[None, :])
    return np.repeat(invc, C, axis=1).astype(np.float32)


_INVC = _fold_counts()


def kernel(vid_in, dists, inds):
    vid = vid_in.reshape(BH, T, C, H, W).transpose(0, 1, 3, 4, 2)
    vpad = jnp.pad(vid, ((0, 0), (0, 0), (0, PS - 1), (0, PS - 1), (0, 0)),
                   mode="reflect")
    vbits = lax.bitcast_convert_type(
        vpad.astype(jnp.bfloat16), jnp.uint16).astype(jnp.uint32)
    vflat = lax.bitcast_convert_type(
        vbits[..., 0::2] | (vbits[..., 1::2] << 16), jnp.int32
    ).reshape(BH, VWORDS // 2)

    ix = inds.reshape(BH, T, H, W, K, 3).astype(jnp.int32)
    tn = ix[..., 0] % T
    bq = ((tn * PADW + ix[..., 1]) * PADW + ix[..., 2]) * (C // 2)
    bq_t = bq.reshape(BH, T, 4, 8, W, K).reshape(NTEC, QPT * K)
    dbits = lax.bitcast_convert_type(
        dists.astype(jnp.bfloat16), jnp.uint16).astype(jnp.uint32)
    dq_t = lax.bitcast_convert_type(
        dbits | (dbits << 16), jnp.int32
    ).reshape(BH, T, 4, 8, W, K).reshape(NTEC, QPT * K)

    parts = _sc_call(vflat, bq_t, dq_t)
    parts = parts.reshape(8, 4, STRIP_ROWS, ROWW)

    res = pl.pallas_call(
        _tc_body,
        out_shape=jax.ShapeDtypeStruct((8, H, W * C), jnp.float32),
    )(parts, jnp.asarray(_INVC))

    out = res.reshape(BH, T, H, W, C).transpose(0, 1, 4, 2, 3)
    return out.reshape(B, HD, T, C, H, W)


# speedup vs baseline: 1.3203x; 1.0054x over previous
import functools

import jax
import jax.numpy as jnp
import numpy as np
from jax import lax
from jax.experimental import pallas as pl
from jax.experimental.pallas import tpu as pltpu
from jax.experimental.pallas import tpu_sc as plsc

PS = 7
B, HD, T, C, H, W = 1, 4, 2, 32, 32, 32
K = 10
BH = B * HD
PADW = W + PS - 1
ROWW = PADW * C
VWORDS = T * PADW * ROWW
STRIP_ROWS = 14
STRIP = STRIP_ROWS * ROWW
NTEC = 32
QPT = 256


def _sc_body(vpad_hbm, bq_hbm, dq_hbm, out_hbm, vpad_v, bq_v, dq_v, opad_v):
    cid = lax.axis_index("c")
    sid = lax.axis_index("s")
    wid = sid * 2 + cid
    bh = wid // 8
    pltpu.sync_copy(vpad_hbm.at[bh], vpad_v)
    pltpu.sync_copy(bq_hbm.at[wid], bq_v)
    pltpu.sync_copy(dq_hbm.at[wid], dq_v)

    iota = jnp.arange(16, dtype=jnp.int32)
    iota2 = iota * 2
    zeros = jnp.zeros((16,), jnp.float32)

    def zbody(i, _):
        plsc.store_scatter(opad_v, [i * 16 + iota], zeros)
        return 0

    lax.fori_loop(0, STRIP // 16, zbody, 0)

    def qbody(q, _):
        nh = q // 32
        nw = q - nh * 32
        obase = nh * ROWW + nw * C
        bvec = bq_v[pl.ds(q * K, 16)]
        wvec = dq_v[pl.ds(q * K, 16)]
        bs = [bvec[k] for k in range(K)]
        ws = [
            plsc.bitcast(jnp.full((16,), wvec[k], jnp.int32), jnp.bfloat16)
            for k in range(K)
        ]

        @plsc.parallel_loop(0, PS)
        def ibody(i):
            roff2 = i * (ROWW // 2)
            for m in range(PS):
                off2 = roff2 + m * 16
                vk = plsc.bitcast(vpad_v[pl.ds(bs[0] + off2, 16)], jnp.bfloat16)
                a0 = vk * ws[0]
                vk = plsc.bitcast(vpad_v[pl.ds(bs[1] + off2, 16)], jnp.bfloat16)
                a1 = vk * ws[1]
                for k in range(2, K, 2):
                    vk = plsc.bitcast(
                        vpad_v[pl.ds(bs[k] + off2, 16)], jnp.bfloat16)
                    a0 = a0 + vk * ws[k]
                    vk = plsc.bitcast(
                        vpad_v[pl.ds(bs[k + 1] + off2, 16)], jnp.bfloat16)
                    a1 = a1 + vk * ws[k + 1]
                lo, hi = plsc.unpack(a0 + a1, format=plsc.PackFormat.INTERLEAVED)
                w0 = obase + i * ROWW + m * 32
                plsc.addupdate_scatter(opad_v, [w0 + iota2], lo)
                plsc.addupdate_scatter(opad_v, [w0 + 1 + iota2], hi)

        return 0

    lax.fori_loop(0, QPT, qbody, 0)
    pltpu.sync_copy(opad_v, out_hbm.at[wid])


_sc_call = functools.partial(
    pl.kernel,
    out_type=jax.ShapeDtypeStruct((NTEC, STRIP), jnp.float32),
    mesh=plsc.VectorSubcoreMesh(core_axis_name="c", subcore_axis_name="s"),
    compiler_params=pltpu.CompilerParams(needs_layout_passes=False),
    scratch_types=[
        pltpu.VMEM((VWORDS // 2,), jnp.int32),
        pltpu.VMEM((QPT * K,), jnp.int32),
        pltpu.VMEM((QPT * K,), jnp.int32),
        pltpu.VMEM((STRIP,), jnp.float32),
    ],
)(_sc_body)


def _tc_body(parts_ref, invc_ref, out_ref):
    p = parts_ref[...]

    def pad_rows(x, lo, hi):
        z = jnp.zeros((8, 1, ROWW), jnp.float32)
        pieces = [z] * lo + [x] + [z] * hi
        return jnp.concatenate(pieces, axis=1) if len(pieces) > 1 else x

    opad = sum(
        pad_rows(p[:, blk], blk * 8, PADW - STRIP_ROWS - blk * 8)
        for blk in range(4)
    )
    rev_rows = jnp.concatenate(
        [opad[:, PADW - 1 - m: PADW - m, :] for m in range(PS - 1)], axis=1
    )
    top = opad[:, :H, :] + pad_rows(rev_rows, 25, 1)
    zc = jnp.zeros((8, H, C), jnp.float32)
    rev_cols = [zc] * 25 + [
        top[:, :, (PADW - 1 - m) * C: (PADW - m) * C] for m in range(PS - 1)
    ] + [zc]
    res = top[:, :, : W * C] + jnp.concatenate(rev_cols, axis=2)
    out_ref[...] = res * invc_ref[...][None, :, :]


def _fold_counts():
    dy = np.arange(PS)
    hh = np.abs(np.arange(H)[:, None] + dy[None, :])
    hh = np.where(hh > H - 1, 2 * (H - 1) - hh, hh)
    ch = np.zeros(H, np.float64)
    np.add.at(ch, hh, 1.0)
    invc = 1.0 / (ch[:, None] * ch[None, :])
    return np.repeat(invc, C, axis=1).astype(np.float32)


_INVC = _fold_counts()


def kernel(vid_in, dists, inds):
    vid = vid_in.reshape(BH, T, C, H, W).transpose(0, 1, 3, 4, 2)
    vpad = jnp.pad(vid, ((0, 0), (0, 0), (0, PS - 1), (0, PS - 1), (0, 0)),
                   mode="reflect")
    vbits = lax.bitcast_convert_type(
        vpad.astype(jnp.bfloat16), jnp.uint16).astype(jnp.uint32)
    vflat = lax.bitcast_convert_type(
        vbits[..., 0::2] | (vbits[..., 1::2] << 16), jnp.int32
    ).reshape(BH, VWORDS // 2)

    ix = inds.reshape(BH, T, H, W, K, 3).astype(jnp.int32)
    tn = ix[..., 0] % T
    bq = ((tn * PADW + ix[..., 1]) * PADW + ix[..., 2]) * (C // 2)
    bq_t = bq.reshape(BH, T, 4, 8, W, K).reshape(NTEC, QPT * K)
    dbits = lax.bitcast_convert_type(
        dists.astype(jnp.bfloat16), jnp.uint16).astype(jnp.uint32)
    dq_t = lax.bitcast_convert_type(
        dbits | (dbits << 16), jnp.int32
    ).reshape(BH, T, 4, 8, W, K).reshape(NTEC, QPT * K)

    parts = _sc_call(vflat, bq_t, dq_t)
    parts = parts.reshape(8, 4, STRIP_ROWS, ROWW)

    res = pl.pallas_call(
        _tc_body,
        out_shape=jax.ShapeDtypeStruct((8, H, W * C), jnp.float32),
    )(parts, jnp.asarray(_INVC))

    out = res.reshape(BH, T, H, W, C).transpose(0, 1, 4, 2, 3)
    return out.reshape(B, HD, T, C, H, W)


# speedup vs baseline: 1.3666x; 1.0350x over previous
import functools

import jax
import jax.numpy as jnp
import numpy as np
from jax import lax
from jax.experimental import pallas as pl
from jax.experimental.pallas import tpu as pltpu
from jax.experimental.pallas import tpu_sc as plsc

PS = 7
B, HD, T, C, H, W = 1, 4, 2, 32, 32, 32
K = 10
BH = B * HD
PADW = W + PS - 1
ROWW = PADW * C
VWORDS = T * PADW * ROWW
STRIP_ROWS = 14
STRIP = STRIP_ROWS * ROWW
NTEC = 32
QPT = 256


def _sc_body(vpad_hbm, bq_hbm, dq_hbm, out_hbm, vpad_v, bq_v, dq_v, opad_v):
    cid = lax.axis_index("c")
    sid = lax.axis_index("s")
    wid = sid * 2 + cid
    bh = wid // 8
    pltpu.sync_copy(vpad_hbm.at[bh], vpad_v)
    pltpu.sync_copy(bq_hbm.at[wid], bq_v)
    pltpu.sync_copy(dq_hbm.at[wid], dq_v)

    iota = jnp.arange(16, dtype=jnp.int32)
    iota2 = iota * 2
    zeros = jnp.zeros((16,), jnp.float32)

    def zbody(i, _):
        plsc.store_scatter(opad_v, [i * 16 + iota], zeros)
        return 0

    lax.fori_loop(0, STRIP // 16, zbody, 0)

    def qbody(q, _):
        nh = q // 32
        nw = q - nh * 32
        obase = nh * ROWW + nw * C
        bvec = bq_v[pl.ds(q * K, 16)]
        wvec = dq_v[pl.ds(q * K, 16)]
        bs = [bvec[k] for k in range(K)]
        ws = [
            plsc.bitcast(jnp.full((16,), wvec[k], jnp.int32), jnp.bfloat16)
            for k in range(K)
        ]

        @plsc.parallel_loop(0, PS)
        def ibody(i):
            roff2 = i * (ROWW // 2)
            for m in range(PS):
                off2 = roff2 + m * 16
                vk = plsc.bitcast(vpad_v[pl.ds(bs[0] + off2, 16)], jnp.bfloat16)
                a0 = vk * ws[0]
                vk = plsc.bitcast(vpad_v[pl.ds(bs[1] + off2, 16)], jnp.bfloat16)
                a1 = vk * ws[1]
                for k in range(2, K, 2):
                    vk = plsc.bitcast(
                        vpad_v[pl.ds(bs[k] + off2, 16)], jnp.bfloat16)
                    a0 = a0 + vk * ws[k]
                    vk = plsc.bitcast(
                        vpad_v[pl.ds(bs[k + 1] + off2, 16)], jnp.bfloat16)
                    a1 = a1 + vk * ws[k + 1]
                lo, hi = plsc.unpack(a0 + a1, format=plsc.PackFormat.INTERLEAVED)
                w0 = obase + i * ROWW + m * 32
                plsc.addupdate_scatter(opad_v, [w0 + iota2], lo)
                plsc.addupdate_scatter(opad_v, [w0 + 1 + iota2], hi)

        return 0

    lax.fori_loop(0, QPT, qbody, 0)
    pltpu.sync_copy(opad_v, out_hbm.at[wid])


_sc_call = functools.partial(
    pl.kernel,
    out_type=jax.ShapeDtypeStruct((NTEC, STRIP), jnp.float32),
    mesh=plsc.VectorSubcoreMesh(core_axis_name="c", subcore_axis_name="s"),
    compiler_params=pltpu.CompilerParams(needs_layout_passes=False),
    scratch_types=[
        pltpu.VMEM((VWORDS // 2,), jnp.int32),
        pltpu.VMEM((QPT * K,), jnp.int32),
        pltpu.VMEM((QPT * K,), jnp.int32),
        pltpu.VMEM((STRIP,), jnp.float32),
    ],
)(_sc_body)


def _tc_body(parts_ref, invc_ref, out_ref):
    p = parts_ref[...]

    def pad_rows(x, lo, hi):
        z = jnp.zeros((8, 1, ROWW), jnp.float32)
        pieces = [z] * lo + [x] + [z] * hi
        return jnp.concatenate(pieces, axis=1) if len(pieces) > 1 else x

    opad = sum(
        pad_rows(p[:, blk], blk * 8, PADW - STRIP_ROWS - blk * 8)
        for blk in range(4)
    )
    rev_rows = jnp.concatenate(
        [opad[:, PADW - 1 - m: PADW - m, :] for m in range(PS - 1)], axis=1
    )
    top = opad[:, :H, :] + pad_rows(rev_rows, 25, 1)
    zc = jnp.zeros((8, H, C), jnp.float32)
    rev_cols = [zc] * 25 + [
        top[:, :, (PADW - 1 - m) * C: (PADW - m) * C] for m in range(PS - 1)
    ] + [zc]
    res = top[:, :, : W * C] + jnp.concatenate(rev_cols, axis=2)
    out_ref[...] = res * invc_ref[...][None, :, :]


def _fold_counts():
    dy = np.arange(PS)
    hh = np.abs(np.arange(H)[:, None] + dy[None, :])
    hh = np.where(hh > H - 1, 2 * (H - 1) - hh, hh)
    ch = np.zeros(H, np.float64)
    np.add.at(ch, hh, 1.0)
    invc = 1.0 / (ch[:, None] * ch[None, :])
    return np.repeat(invc, C, axis=1).astype(np.float32)


_INVC = _fold_counts()


def kernel(vid_in, dists, inds):
    vbits = lax.bitcast_convert_type(
        vid_in.reshape(BH, T, C, H, W).astype(jnp.bfloat16), jnp.uint16
    ).astype(jnp.uint32)
    vpk = lax.bitcast_convert_type(
        vbits[:, :, 0::2] | (vbits[:, :, 1::2] << 16), jnp.int32
    ).transpose(0, 1, 3, 4, 2)
    vflat = jnp.pad(
        vpk, ((0, 0), (0, 0), (0, PS - 1), (0, PS - 1), (0, 0)),
        mode="reflect").reshape(BH, VWORDS // 2)

    ix = inds.reshape(BH, T, H, W, K, 3).astype(jnp.int32)
    tn = ix[..., 0] % T
    bq = ((tn * PADW + ix[..., 1]) * PADW + ix[..., 2]) * (C // 2)
    bq_t = bq.reshape(BH, T, 4, 8, W, K).reshape(NTEC, QPT * K)
    dbits = lax.bitcast_convert_type(
        dists.astype(jnp.bfloat16), jnp.uint16).astype(jnp.uint32)
    dq_t = lax.bitcast_convert_type(
        dbits | (dbits << 16), jnp.int32
    ).reshape(BH, T, 4, 8, W, K).reshape(NTEC, QPT * K)

    parts = _sc_call(vflat, bq_t, dq_t)
    parts = parts.reshape(8, 4, STRIP_ROWS, ROWW)

    res = pl.pallas_call(
        _tc_body,
        out_shape=jax.ShapeDtypeStruct((8, H, W * C), jnp.float32),
    )(parts, jnp.asarray(_INVC))

    out = res.reshape(BH, T, H, W, C).transpose(0, 1, 4, 2, 3)
    return out.reshape(B, HD, T, C, H, W)


# speedup vs baseline: 1.3818x; 1.0112x over previous
import functools

import jax
import jax.numpy as jnp
import numpy as np
from jax import lax
from jax.experimental import pallas as pl
from jax.experimental.pallas import tpu as pltpu
from jax.experimental.pallas import tpu_sc as plsc

PS = 7
B, HD, T, C, H, W = 1, 4, 2, 32, 32, 32
K = 10
BH = B * HD
PADW = W + PS - 1
ROWW = PADW * C
VWORDS = T * PADW * ROWW
STRIP_ROWS = 14
STRIP = STRIP_ROWS * ROWW
NTEC = 32
QPT = 256


def _sc_body(vpad_hbm, bq_hbm, dq_hbm, out_hbm, vpad_v, bq_v, dq_v, opad_v):
    cid = lax.axis_index("c")
    sid = lax.axis_index("s")
    wid = sid * 2 + cid
    bh = wid // 8
    pltpu.sync_copy(vpad_hbm.at[bh], vpad_v)
    pltpu.sync_copy(bq_hbm.at[wid], bq_v)
    pltpu.sync_copy(dq_hbm.at[wid], dq_v)

    iota = jnp.arange(16, dtype=jnp.int32)
    iota2 = iota * 2
    zeros = jnp.zeros((16,), jnp.float32)

    def zbody(i, _):
        plsc.store_scatter(opad_v, [i * 16 + iota], zeros)
        return 0

    lax.fori_loop(0, STRIP // 16, zbody, 0)

    def qbody(q, _):
        nh = q // 32
        nw = q - nh * 32
        obase = nh * ROWW + nw * C
        bvec = bq_v[pl.ds(q * K, 16)]
        wvec = dq_v[pl.ds(q * K, 16)]
        bs = [bvec[k] for k in range(K)]
        ws = [
            plsc.bitcast(jnp.full((16,), wvec[k], jnp.int32), jnp.bfloat16)
            for k in range(K)
        ]

        @plsc.parallel_loop(0, PS)
        def ibody(i):
            roff2 = i * (ROWW // 2)
            for m in range(PS):
                off2 = roff2 + m * 16
                vk = plsc.bitcast(vpad_v[pl.ds(bs[0] + off2, 16)], jnp.bfloat16)
                a0 = vk * ws[0]
                vk = plsc.bitcast(vpad_v[pl.ds(bs[1] + off2, 16)], jnp.bfloat16)
                a1 = vk * ws[1]
                for k in range(2, K, 2):
                    vk = plsc.bitcast(
                        vpad_v[pl.ds(bs[k] + off2, 16)], jnp.bfloat16)
                    a0 = a0 + vk * ws[k]
                    vk = plsc.bitcast(
                        vpad_v[pl.ds(bs[k + 1] + off2, 16)], jnp.bfloat16)
                    a1 = a1 + vk * ws[k + 1]
                lo, hi = plsc.unpack(a0 + a1, format=plsc.PackFormat.INTERLEAVED)
                w0 = obase + i * ROWW + m * 32
                plsc.addupdate(opad_v.at[pl.ds(w0, 16)], lo)
                plsc.addupdate(opad_v.at[pl.ds(w0 + 16, 16)], hi)

        return 0

    lax.fori_loop(0, QPT, qbody, 0)
    pltpu.sync_copy(opad_v, out_hbm.at[wid])


_sc_call = functools.partial(
    pl.kernel,
    out_type=jax.ShapeDtypeStruct((NTEC, STRIP), jnp.float32),
    mesh=plsc.VectorSubcoreMesh(core_axis_name="c", subcore_axis_name="s"),
    compiler_params=pltpu.CompilerParams(needs_layout_passes=False),
    scratch_types=[
        pltpu.VMEM((VWORDS // 2,), jnp.int32),
        pltpu.VMEM((QPT * K,), jnp.int32),
        pltpu.VMEM((QPT * K,), jnp.int32),
        pltpu.VMEM((STRIP,), jnp.float32),
    ],
)(_sc_body)


def _tc_body(parts_ref, invc_ref, out_ref):
    p = parts_ref[...]

    def pad_rows(x, lo, hi):
        z = jnp.zeros((8, 1, ROWW), jnp.float32)
        pieces = [z] * lo + [x] + [z] * hi
        return jnp.concatenate(pieces, axis=1) if len(pieces) > 1 else x

    opad = sum(
        pad_rows(p[:, blk], blk * 8, PADW - STRIP_ROWS - blk * 8)
        for blk in range(4)
    )
    rev_rows = jnp.concatenate(
        [opad[:, PADW - 1 - m: PADW - m, :] for m in range(PS - 1)], axis=1
    )
    top = opad[:, :H, :] + pad_rows(rev_rows, 25, 1)
    zc = jnp.zeros((8, H, C), jnp.float32)
    rev_cols = [zc] * 25 + [
        top[:, :, (PADW - 1 - m) * C: (PADW - m) * C] for m in range(PS - 1)
    ] + [zc]
    res = top[:, :, : W * C] + jnp.concatenate(rev_cols, axis=2)
    out_ref[...] = res * invc_ref[...][None, :, :]


def _fold_counts():
    dy = np.arange(PS)
    hh = np.abs(np.arange(H)[:, None] + dy[None, :])
    hh = np.where(hh > H - 1, 2 * (H - 1) - hh, hh)
    ch = np.zeros(H, np.float64)
    np.add.at(ch, hh, 1.0)
    invc = 1.0 / (ch[:, None] * ch[None, :])
    return np.repeat(invc, C, axis=1).astype(np.float32)


_INVC = _fold_counts()


def kernel(vid_in, dists, inds):
    vbits = lax.bitcast_convert_type(
        vid_in.reshape(BH, T, C, H, W).astype(jnp.bfloat16), jnp.uint16
    ).astype(jnp.uint32)
    vpk = lax.bitcast_convert_type(
        vbits[:, :, : C // 2] | (vbits[:, :, C // 2:] << 16), jnp.int32
    ).transpose(0, 1, 3, 4, 2)
    vflat = jnp.pad(
        vpk, ((0, 0), (0, 0), (0, PS - 1), (0, PS - 1), (0, 0)),
        mode="reflect").reshape(BH, VWORDS // 2)

    ix = inds.reshape(BH, T, H, W, K, 3).astype(jnp.int32)
    tn = ix[..., 0] % T
    bq = ((tn * PADW + ix[..., 1]) * PADW + ix[..., 2]) * (C // 2)
    bq_t = bq.reshape(BH, T, 4, 8, W, K).reshape(NTEC, QPT * K)
    dbits = lax.bitcast_convert_type(
        dists.astype(jnp.bfloat16), jnp.uint16).astype(jnp.uint32)
    dq_t = lax.bitcast_convert_type(
        dbits | (dbits << 16), jnp.int32
    ).reshape(BH, T, 4, 8, W, K).reshape(NTEC, QPT * K)

    parts = _sc_call(vflat, bq_t, dq_t)
    parts = parts.reshape(8, 4, STRIP_ROWS, ROWW)

    res = pl.pallas_call(
        _tc_body,
        out_shape=jax.ShapeDtypeStruct((8, H, W * C), jnp.float32),
    )(parts, jnp.asarray(_INVC))

    out = res.reshape(BH, T, H, W, C).transpose(0, 1, 4, 2, 3)
    return out.reshape(B, HD, T, C, H, W)


# speedup vs baseline: 1.3821x; 1.0002x over previous
import functools

import jax
import jax.numpy as jnp
import numpy as np
from jax import lax
from jax.experimental import pallas as pl
from jax.experimental.pallas import tpu as pltpu
from jax.experimental.pallas import tpu_sc as plsc

PS = 7
B, HD, T, C, H, W = 1, 4, 2, 32, 32, 32
K = 10
BH = B * HD
PADW = W + PS - 1
ROWW = PADW * C
VWORDS = T * PADW * ROWW
STRIP_ROWS = 14
STRIP = STRIP_ROWS * ROWW
NTEC = 32
QPT = 256


def _sc_body(vpad_hbm, bq_hbm, dq_hbm, out_hbm, vpad_v, bq_v, dq_v, opad_v):
    cid = lax.axis_index("c")
    sid = lax.axis_index("s")
    wid = sid * 2 + cid
    bh = wid // 8
    pltpu.sync_copy(vpad_hbm.at[bh], vpad_v)
    pltpu.sync_copy(bq_hbm.at[wid], bq_v)
    pltpu.sync_copy(dq_hbm.at[wid], dq_v)

    iota = jnp.arange(16, dtype=jnp.int32)
    zeros = jnp.zeros((16,), jnp.float32)

    def zbody(i, _):
        plsc.store_scatter(opad_v, [i * 16 + iota], zeros)
        return 0

    lax.fori_loop(0, STRIP // 16, zbody, 0)

    def qbody(q, _):
        nh = q // 32
        nw = q - nh * 32
        obase = nh * ROWW + nw * C
        bvec = bq_v[pl.ds(q * K, 16)]
        wvec = dq_v[pl.ds(q * K, 16)]
        bs = [bvec[k] for k in range(K)]
        ws = [
            plsc.bitcast(jnp.full((16,), wvec[k], jnp.int32), jnp.bfloat16)
            for k in range(K)
        ]

        @plsc.parallel_loop(0, PS)
        def ibody(i):
            roff2 = i * (ROWW // 2)
            for m in range(PS):
                off2 = roff2 + m * 16
                vk = plsc.bitcast(vpad_v[pl.ds(bs[0] + off2, 16)], jnp.bfloat16)
                a0 = vk * ws[0]
                vk = plsc.bitcast(vpad_v[pl.ds(bs[1] + off2, 16)], jnp.bfloat16)
                a1 = vk * ws[1]
                for k in range(2, K, 2):
                    vk = plsc.bitcast(
                        vpad_v[pl.ds(bs[k] + off2, 16)], jnp.bfloat16)
                    a0 = a0 + vk * ws[k]
                    vk = plsc.bitcast(
                        vpad_v[pl.ds(bs[k + 1] + off2, 16)], jnp.bfloat16)
                    a1 = a1 + vk * ws[k + 1]
                lo, hi = plsc.unpack(a0 + a1, format=plsc.PackFormat.INTERLEAVED)
                w0 = obase + i * ROWW + m * 32
                plsc.addupdate(opad_v.at[pl.ds(w0, 16)], lo)
                plsc.addupdate(opad_v.at[pl.ds(w0 + 16, 16)], hi)

        return 0

    lax.fori_loop(0, QPT, qbody, 0)
    pltpu.sync_copy(opad_v, out_hbm.at[wid])


_sc_call = functools.partial(
    pl.kernel,
    out_type=jax.ShapeDtypeStruct((NTEC, STRIP), jnp.float32),
    mesh=plsc.VectorSubcoreMesh(core_axis_name="c", subcore_axis_name="s"),
    compiler_params=pltpu.CompilerParams(needs_layout_passes=False),
    scratch_types=[
        pltpu.VMEM((VWORDS // 2,), jnp.int32),
        pltpu.VMEM((QPT * K,), jnp.int32),
        pltpu.VMEM((QPT * K,), jnp.int32),
        pltpu.VMEM((STRIP,), jnp.float32),
    ],
)(_sc_body)


def _tc_body(parts_ref, invc_ref, out_ref):
    p = parts_ref[...]

    def pad_rows(x, lo, hi):
        z = jnp.zeros((8, 1, ROWW), jnp.float32)
        pieces = [z] * lo + [x] + [z] * hi
        return jnp.concatenate(pieces, axis=1) if len(pieces) > 1 else x

    opad = sum(
        pad_rows(p[:, blk], blk * 8, PADW - STRIP_ROWS - blk * 8)
        for blk in range(4)
    )
    rev_rows = jnp.concatenate(
        [opad[:, PADW - 1 - m: PADW - m, :] for m in range(PS - 1)], axis=1
    )
    top = opad[:, :H, :] + pad_rows(rev_rows, 25, 1)
    zc = jnp.zeros((8, H, C), jnp.float32)
    rev_cols = [zc] * 25 + [
        top[:, :, (PADW - 1 - m) * C: (PADW - m) * C] for m in range(PS - 1)
    ] + [zc]
    res = top[:, :, : W * C] + jnp.concatenate(rev_cols, axis=2)
    out_ref[...] = res * invc_ref[...][None, :, :]


def _fold_counts():
    dy = np.arange(PS)
    hh = np.abs(np.arange(H)[:, None] + dy[None, :])
    hh = np.where(hh > H - 1, 2 * (H - 1) - hh, hh)
    ch = np.zeros(H, np.float64)
    np.add.at(ch, hh, 1.0)
    invc = 1.0 / (ch[:, None] * ch[None, :])
    return np.repeat(invc, C, axis=1).astype(np.float32)


_INVC = _fold_counts()


def kernel(vid_in, dists, inds):
    vbits = lax.bitcast_convert_type(
        vid_in.reshape(BH, T, C, H, W).astype(jnp.bfloat16), jnp.uint16
    ).astype(jnp.uint32)
    vpk = lax.bitcast_convert_type(
        vbits[:, :, : C // 2] | (vbits[:, :, C // 2:] << 16), jnp.int32
    ).transpose(0, 1, 3, 4, 2)
    vflat = jnp.pad(
        vpk, ((0, 0), (0, 0), (0, PS - 1), (0, PS - 1), (0, 0)),
        mode="reflect").reshape(BH, VWORDS // 2)

    ix = inds.reshape(BH, T, H, W, K, 3).astype(jnp.int32)
    tn = ix[..., 0] % T
    bq = ((tn * PADW + ix[..., 1]) * PADW + ix[..., 2]) * (C // 2)
    bq_t = bq.reshape(BH, T, 4, 8, W, K).reshape(NTEC, QPT * K)
    dbits = lax.bitcast_convert_type(
        dists.astype(jnp.bfloat16), jnp.uint16).astype(jnp.uint32)
    dq_t = lax.bitcast_convert_type(
        dbits | (dbits << 16), jnp.int32
    ).reshape(BH, T, 4, 8, W, K).reshape(NTEC, QPT * K)

    parts = _sc_call(vflat, bq_t, dq_t)
    parts = parts.reshape(8, 4, STRIP_ROWS, ROWW)

    res = pl.pallas_call(
        _tc_body,
        out_shape=jax.ShapeDtypeStruct((8, H, W * C), jnp.float32),
    )(parts, jnp.asarray(_INVC))

    out = res.reshape(BH, T, H, W, C).transpose(0, 1, 4, 2, 3)
    return out.reshape(B, HD, T, C, H, W)
